# baseline (speedup 1.0000x reference)
import functools

import jax
import jax.numpy as jnp
import numpy as np
from jax import lax
from jax.experimental import pallas as pl
from jax.experimental.pallas import tpu as pltpu
from jax.experimental.pallas import tpu_sc as plsc

NC = 2
NS = 16
GRP = 64
UNIT = 8 * GRP


def _mesh():
    return plsc.VectorSubcoreMesh(core_axis_name="c", subcore_axis_name="s")


def _node_split(n, w):
    blocks = n // 8
    q, rr = blocks // NS, blocks % NS
    start = 8 * (w * q + jnp.minimum(w, rr))
    return start, 8 * q, w < rr


def _deg_call(n, u_total):
    u_per = u_total // (NC * NS)

    @functools.partial(
        pl.kernel,
        out_type=jax.ShapeDtypeStruct((NC, n, 16), jnp.float32),
        mesh=_mesh(),
        scratch_types=[
            pltpu.VMEM_SHARED((n + 8, 16), jnp.float32),
            pltpu.VMEM((8 * (n // 8 // NS + 1), 16), jnp.float32),
            pltpu.VMEM((GRP, 16), jnp.float32),
            pltpu.VMEM((8, GRP), jnp.int32),
        ],
    )
    def deg_kernel(dst_hbm, out_hbm, acc, zbuf, ones, idx):
        c = lax.axis_index("c")
        s = lax.axis_index("s")
        w = c * NS + s
        zrows = zbuf.shape[0]

        def fill(i, _):
            zbuf[i] = jnp.zeros((16,), jnp.float32)
            return 0

        lax.fori_loop(0, zrows, fill, 0)

        def fill1(i, _):
            ones[i] = jnp.ones((16,), jnp.float32)
            return 0

        lax.fori_loop(0, GRP, fill1, 0)

        start, main, extra = _node_split(n, s)
        pltpu.sync_copy(zbuf.at[pl.ds(0, main)], acc.at[pl.ds(start, main)])

        @pl.when(extra)
        def _():
            pltpu.sync_copy(zbuf.at[pl.ds(0, 8)],
                            acc.at[pl.ds(start + main, 8)])

        @pl.when(s == 0)
        def _():
            pltpu.sync_copy(zbuf.at[pl.ds(0, 8)], acc.at[pl.ds(n, 8)])

        plsc.subcore_barrier()

        def body(i, _):
            pltpu.sync_copy(dst_hbm.at[w * u_per + i], idx)
            for j in range(8):
                pltpu.sync_copy(ones, acc.at[idx.at[j]], add=True)
            return 0

        lax.fori_loop(0, u_per, body, 0)
        plsc.subcore_barrier()
        pltpu.sync_copy(acc.at[pl.ds(start, main)],
                        out_hbm.at[c, pl.ds(start, main)])

        @pl.when(extra)
        def _():
            pltpu.sync_copy(acc.at[pl.ds(start + main, 8)],
                            out_hbm.at[c, pl.ds(start + main, 8)])

    return deg_kernel


NSLOT = 4
AHEAD = 2


def _prop_call(n, nu):
    @functools.partial(
        pl.kernel,
        out_type=jax.ShapeDtypeStruct((NC, n, 128), jnp.float32),
        mesh=_mesh(),
        compiler_params=pltpu.CompilerParams(use_tc_tiling_on_sc=False),
        scratch_types=[
            pltpu.VMEM_SHARED((n + 8, 128), jnp.float32),
            pltpu.VMEM((2, 8, GRP), jnp.int32),
            pltpu.VMEM((2, 8, GRP), jnp.int32),
            [pltpu.VMEM((GRP, 128), jnp.float32)] * NSLOT,
            [pltpu.SemaphoreType.DMA] * NSLOT,
            [pltpu.SemaphoreType.DMA] * NSLOT,
            pltpu.SemaphoreType.DMA,
            pltpu.SemaphoreType.DMA,
        ],
    )
    def prop_kernel(y_hbm, src_hbm, dst_hbm, out_hbm,
                    acc, idx_s, idx_d, rows, sg, ss, sis, sid):
        c = lax.axis_index("c")
        s = lax.axis_index("s")
        g_total = 8 * nu
        start, main, extra = _node_split(n, s)
        pltpu.sync_copy(y_hbm.at[c, pl.ds(start, main)],
                        acc.at[pl.ds(start, main)])

        @pl.when(extra)
        def _():
            pltpu.sync_copy(y_hbm.at[c, pl.ds(start + main, 8)],
                            acc.at[pl.ds(start + main, 8)])

        pltpu.sync_copy(src_hbm.at[s, 0], idx_s.at[0])
        pltpu.sync_copy(dst_hbm.at[s, 0], idx_d.at[0])
        plsc.subcore_barrier()

        def gather(p, j, slot):
            pltpu.async_copy(y_hbm.at[c].at[idx_s.at[p, j]], rows[slot],
                             sg[slot])

        def wait_gather(p, j, slot):
            pltpu.make_async_copy(y_hbm.at[c].at[idx_s.at[p, j]], rows[slot],
                                  sg[slot]).wait()

        def wait_scatter(p, j, slot):
            pltpu.make_async_copy(rows[slot], acc.at[idx_d.at[p, j]],
                                  ss[slot]).wait()

        for g in range(AHEAD):
            gather(0, g, g)

        def body(u, _):
            p = u % 2

            @pl.when(u + 1 < nu)
            def _():
                pltpu.async_copy(src_hbm.at[s, u + 1], idx_s.at[1 - p], sis)
                pltpu.async_copy(dst_hbm.at[s, u + 1], idx_d.at[1 - p], sid)

            for j in range(8):
                g = 8 * u + j
                k = j % NSLOT
                k2 = (j + AHEAD) % NSLOT
                wait_gather(p, j, k)
                pltpu.async_copy(rows[k], acc.at[idx_d.at[p, j]], ss[k],
                                 add=True)

                @pl.when(g >= NSLOT - AHEAD)
                def _():
                    wait_scatter(p, j, k2)

                ja = j + AHEAD
                if ja >= 8:
                    @pl.when(u + 1 < nu)
                    def _():
                        if ja == 8:
                            pltpu.make_async_copy(src_hbm.at[s, u],
                                                  idx_s.at[p], sis).wait()
                            pltpu.make_async_copy(dst_hbm.at[s, u],
                                                  idx_d.at[p], sid).wait()
                        gather(1 - p, ja - 8, k2)
                else:
                    @pl.when(g + AHEAD < g_total)
                    def _():
                        gather(p, ja, k2)
            return 0

        lax.fori_loop(0, nu, body, 0)
        for i in range(NSLOT - AHEAD):
            slot = (g_total - (NSLOT - AHEAD) + i) % NSLOT
            wait_scatter((nu - 1) % 2, 0, slot)
        plsc.subcore_barrier()
        pltpu.sync_copy(acc.at[pl.ds(start, main)],
                        out_hbm.at[c, pl.ds(start, main)])

        @pl.when(extra)
        def _():
            pltpu.sync_copy(acc.at[pl.ds(start + main, 8)],
                            out_hbm.at[c, pl.ds(start + main, 8)])

    return prop_kernel


def _dinv_of(deg_ref):
    d = deg_ref[0, :, 0:1] + deg_ref[1, :, 0:1] + 1.0
    return lax.rsqrt(d)


def _tc_pre_body(x_ref, wp_ref, bp_ref, w1_ref, deg_ref, y_ref):
    h0 = jnp.dot(x_ref[...], wp_ref[...], preferred_element_type=jnp.float32)
    h0 = jnp.maximum(h0 + bp_ref[...], 0.0)
    dinv = _dinv_of(deg_ref)
    y = jnp.dot(h0, w1_ref[...], preferred_element_type=jnp.float32) * dinv
    y_ref[0] = y[:, :128]
    y_ref[1] = y[:, 128:]


def _tc_mid_body(s_ref, deg_ref, g_ref, beta_ref, w_ref, y_ref):
    dinv = _dinv_of(deg_ref)
    s_cat = jnp.concatenate([s_ref[0], s_ref[1]], axis=1)
    h = jnp.maximum(s_cat * dinv * g_ref[...] + beta_ref[...], 0.0)
    y = jnp.dot(h, w_ref[...], preferred_element_type=jnp.float32) * dinv
    y_ref[0] = y[:, :128]
    y_ref[1] = y[:, 128:]


def _tc_post_body(s_ref, deg_ref, g3_ref, beta3_ref, wo1_ref, g4_ref,
                  beta4_ref, wo2_ref, bo2_ref, out_ref):
    dinv = _dinv_of(deg_ref)
    s_cat = jnp.concatenate([s_ref[0], s_ref[1]], axis=1)
    h = jnp.maximum(s_cat * dinv * g3_ref[...] + beta3_ref[...], 0.0)
    h4 = jnp.dot(h, wo1_ref[...], preferred_element_type=jnp.float32)
    h4 = jnp.maximum(h4 * g4_ref[...] + beta4_ref[...], 0.0)
    lg = jnp.dot(h4, wo2_ref[...], preferred_element_type=jnp.float32)
    lg = lg + bo2_ref[...]
    m = jnp.max(lg, axis=1, keepdims=True)
    lse = m + jnp.log(jnp.sum(jnp.exp(lg - m), axis=1, keepdims=True))
    out_ref[...] = lg - lse


def _row(r, h):
    return pl.BlockSpec((r, h), lambda i: (i, 0))


def _fixed(*shape):
    nd = len(shape)
    return pl.BlockSpec(shape, lambda i, _n=nd: (0,) * _n)


def _half(r):
    return pl.BlockSpec((NC, r, 128), lambda i: (0, i, 0))


def _degspec(r):
    return pl.BlockSpec((NC, r, 16), lambda i: (0, i, 0))


def kernel(x, edge_index, Wp, bp, W1, b1, W2, b2, W3, b3, Wo1, bo1, Wo2, bo2,
           gn1, bnb1, gn2, bnb2, gn3, bnb3, gn4, bnb4):
    n, f_in = x.shape
    h_dim = W1.shape[0]
    e = edge_index.shape[1]
    c_dim = Wo2.shape[1]
    assert n % 8 == 0 and h_dim == 256

    e2 = -(-e // (NS * UNIT)) * (NS * UNIT)
    pad = e2 - e
    src_p = jnp.concatenate(
        [edge_index[0], jnp.zeros((pad,), edge_index.dtype)])
    dst_p = jnp.concatenate(
        [edge_index[1], n + (jnp.arange(pad, dtype=edge_index.dtype) % 8)])
    dst3 = dst_p.reshape(e2 // UNIT, 8, GRP)
    u_total = e2 // UNIT
    nu = e2 // (NS * UNIT)
    src5 = src_p.reshape(NS, nu, 8, GRP)
    dst5 = dst_p.reshape(NS, nu, 8, GRP)

    sc = 1.0 / np.sqrt(1.0 + 1e-5)
    g1 = (gn1 * sc).reshape(1, h_dim)
    g2 = (gn2 * sc).reshape(1, h_dim)
    g3 = (gn3 * sc).reshape(1, h_dim)
    g4 = (gn4 * sc).reshape(1, h_dim)
    beta1 = (b1 * g1[0] + bnb1).reshape(1, h_dim)
    beta2 = (b2 * g2[0] + bnb2).reshape(1, h_dim)
    beta3 = (b3 * g3[0] + bnb3).reshape(1, h_dim)
    beta4 = (bo1 * g4[0] + bnb4).reshape(1, h_dim)

    deg = _deg_call(n, u_total)(dst3)

    r = 2000
    grid = (n // r,)

    y1 = pl.pallas_call(
        _tc_pre_body,
        grid=grid,
        in_specs=[_row(r, f_in), _fixed(f_in, h_dim), _fixed(1, h_dim),
                  _fixed(h_dim, h_dim), _degspec(r)],
        out_specs=_half(r),
        out_shape=jax.ShapeDtypeStruct((NC, n, 128), jnp.float32),
    )(x, Wp, bp.reshape(1, h_dim), W1, deg)

    prop = _prop_call(n, nu)
    s1 = prop(y1, src5, dst5)

    def mid(s_in, g, beta, w):
        return pl.pallas_call(
            _tc_mid_body,
            grid=grid,
            in_specs=[_half(r), _degspec(r), _fixed(1, h_dim),
                      _fixed(1, h_dim), _fixed(h_dim, h_dim)],
            out_specs=_half(r),
            out_shape=jax.ShapeDtypeStruct((NC, n, 128), jnp.float32),
        )(s_in, deg, g, beta, w)

    y2 = mid(s1, g1, beta1, W2)
    s2 = prop(y2, src5, dst5)
    y3 = mid(s2, g2, beta2, W3)
    s3 = prop(y3, src5, dst5)

    out = pl.pallas_call(
        _tc_post_body,
        grid=grid,
        in_specs=[_half(r), _degspec(r), _fixed(1, h_dim), _fixed(1, h_dim),
                  _fixed(h_dim, h_dim), _fixed(1, h_dim), _fixed(1, h_dim),
                  _fixed(h_dim, c_dim), _fixed(1, c_dim)],
        out_specs=_row(r, c_dim),
        out_shape=jax.ShapeDtypeStruct((n, c_dim), jnp.float32),
    )(s3, deg, g3, beta3, Wo1, g4, beta4, Wo2, bo2.reshape(1, c_dim))
    return out


# speedup vs baseline: 1.2013x; 1.2013x over previous
import functools

import jax
import jax.numpy as jnp
import numpy as np
from jax import lax
from jax.experimental import pallas as pl
from jax.experimental.pallas import tpu as pltpu
from jax.experimental.pallas import tpu_sc as plsc

NC = 2
NS = 16
GRP = 64
UNIT = 8 * GRP


def _mesh():
    return plsc.VectorSubcoreMesh(core_axis_name="c", subcore_axis_name="s")


def _node_split(n, w):
    blocks = n // 8
    q, rr = blocks // NS, blocks % NS
    start = 8 * (w * q + jnp.minimum(w, rr))
    return start, 8 * q, w < rr


def _deg_call(n, u_total):
    u_per = u_total // (NC * NS)

    @functools.partial(
        pl.kernel,
        out_type=jax.ShapeDtypeStruct((NC, n, 16), jnp.float32),
        mesh=_mesh(),
        scratch_types=[
            pltpu.VMEM_SHARED((n + 8, 16), jnp.float32),
            pltpu.VMEM((8 * (n // 8 // NS + 1), 16), jnp.float32),
            pltpu.VMEM((GRP, 16), jnp.float32),
            pltpu.VMEM((8, GRP), jnp.int32),
        ],
    )
    def deg_kernel(dst_hbm, out_hbm, acc, zbuf, ones, idx):
        c = lax.axis_index("c")
        s = lax.axis_index("s")
        w = c * NS + s
        zrows = zbuf.shape[0]

        def fill(i, _):
            zbuf[i] = jnp.zeros((16,), jnp.float32)
            return 0

        lax.fori_loop(0, zrows, fill, 0)

        def fill1(i, _):
            ones[i] = jnp.ones((16,), jnp.float32)
            return 0

        lax.fori_loop(0, GRP, fill1, 0)

        start, main, extra = _node_split(n, s)
        pltpu.sync_copy(zbuf.at[pl.ds(0, main)], acc.at[pl.ds(start, main)])

        @pl.when(extra)
        def _():
            pltpu.sync_copy(zbuf.at[pl.ds(0, 8)],
                            acc.at[pl.ds(start + main, 8)])

        @pl.when(s == 0)
        def _():
            pltpu.sync_copy(zbuf.at[pl.ds(0, 8)], acc.at[pl.ds(n, 8)])

        plsc.subcore_barrier()

        def body(i, _):
            pltpu.sync_copy(dst_hbm.at[w * u_per + i], idx)
            for j in range(8):
                pltpu.sync_copy(ones, acc.at[idx.at[j]], add=True)
            return 0

        lax.fori_loop(0, u_per, body, 0)
        plsc.subcore_barrier()
        pltpu.sync_copy(acc.at[pl.ds(start, main)],
                        out_hbm.at[c, pl.ds(start, main)])

        @pl.when(extra)
        def _():
            pltpu.sync_copy(acc.at[pl.ds(start + main, 8)],
                            out_hbm.at[c, pl.ds(start + main, 8)])

    return deg_kernel


NSLOT = 4
AHEAD = 3


def _prop_call(n, nu):
    @functools.partial(
        pl.kernel,
        out_type=jax.ShapeDtypeStruct((NC, n, 128), jnp.float32),
        mesh=_mesh(),
        scratch_types=[
            pltpu.VMEM_SHARED((n + 8, 128), jnp.float32),
            pltpu.VMEM((2, 8, GRP), jnp.int32),
            pltpu.VMEM((2, 8, GRP), jnp.int32),
            [pltpu.VMEM((GRP, 128), jnp.float32)] * NSLOT,
            [pltpu.SemaphoreType.DMA] * NSLOT,
            [pltpu.SemaphoreType.DMA] * NSLOT,
            pltpu.SemaphoreType.DMA,
            pltpu.SemaphoreType.DMA,
        ],
    )
    def prop_kernel(y_hbm, src_hbm, dst_hbm, out_hbm,
                    acc, idx_s, idx_d, rows, sg, ss, sis, sid):
        c = lax.axis_index("c")
        s = lax.axis_index("s")
        g_total = 8 * nu
        start, main, extra = _node_split(n, s)
        pltpu.sync_copy(y_hbm.at[c, pl.ds(start, main)],
                        acc.at[pl.ds(start, main)])

        @pl.when(extra)
        def _():
            pltpu.sync_copy(y_hbm.at[c, pl.ds(start + main, 8)],
                            acc.at[pl.ds(start + main, 8)])

        pltpu.sync_copy(src_hbm.at[s, 0], idx_s.at[0])
        pltpu.sync_copy(dst_hbm.at[s, 0], idx_d.at[0])
        plsc.subcore_barrier()

        def gather(p, j, slot):
            pltpu.async_copy(y_hbm.at[c].at[idx_s.at[p, j]], rows[slot],
                             sg[slot])

        def wait_gather(p, j, slot):
            pltpu.make_async_copy(y_hbm.at[c].at[idx_s.at[p, j]], rows[slot],
                                  sg[slot]).wait()

        def wait_scatter(p, j, slot):
            pltpu.make_async_copy(rows[slot], acc.at[idx_d.at[p, j]],
                                  ss[slot]).wait()

        for g in range(AHEAD):
            gather(0, g, g)

        def body(u, _):
            p = u % 2

            @pl.when(u + 1 < nu)
            def _():
                pltpu.async_copy(src_hbm.at[s, u + 1], idx_s.at[1 - p], sis)
                pltpu.async_copy(dst_hbm.at[s, u + 1], idx_d.at[1 - p], sid)

            for j in range(8):
                g = 8 * u + j
                k = j % NSLOT
                k2 = (j + AHEAD) % NSLOT
                wait_gather(p, j, k)
                pltpu.async_copy(rows[k], acc.at[idx_d.at[p, j]], ss[k],
                                 add=True)

                @pl.when(g >= NSLOT - AHEAD)
                def _():
                    wait_scatter(p, j, k2)

                ja = j + AHEAD
                if ja >= 8:
                    @pl.when(u + 1 < nu)
                    def _():
                        if ja == 8:
                            pltpu.make_async_copy(src_hbm.at[s, u],
                                                  idx_s.at[p], sis).wait()
                            pltpu.make_async_copy(dst_hbm.at[s, u],
                                                  idx_d.at[p], sid).wait()
                        gather(1 - p, ja - 8, k2)
                else:
                    @pl.when(g + AHEAD < g_total)
                    def _():
                        gather(p, ja, k2)
            return 0

        lax.fori_loop(0, nu, body, 0)
        for i in range(NSLOT - AHEAD):
            slot = (g_total - (NSLOT - AHEAD) + i) % NSLOT
            wait_scatter((nu - 1) % 2, 0, slot)
        plsc.subcore_barrier()
        pltpu.sync_copy(acc.at[pl.ds(start, main)],
                        out_hbm.at[c, pl.ds(start, main)])

        @pl.when(extra)
        def _():
            pltpu.sync_copy(acc.at[pl.ds(start + main, 8)],
                            out_hbm.at[c, pl.ds(start + main, 8)])

    return prop_kernel


def _dinv_of(deg_ref):
    d = deg_ref[0, :, 0:1] + deg_ref[1, :, 0:1] + 1.0
    return lax.rsqrt(d)


def _tc_pre_body(x_ref, wp_ref, bp_ref, w1_ref, deg_ref, y_ref):
    h0 = jnp.dot(x_ref[...], wp_ref[...], preferred_element_type=jnp.float32)
    h0 = jnp.maximum(h0 + bp_ref[...], 0.0)
    dinv = _dinv_of(deg_ref)
    y = jnp.dot(h0, w1_ref[...], preferred_element_type=jnp.float32) * dinv
    y_ref[0] = y[:, :128]
    y_ref[1] = y[:, 128:]


def _tc_mid_body(s_ref, deg_ref, g_ref, beta_ref, w_ref, y_ref):
    dinv = _dinv_of(deg_ref)
    s_cat = jnp.concatenate([s_ref[0], s_ref[1]], axis=1)
    h = jnp.maximum(s_cat * dinv * g_ref[...] + beta_ref[...], 0.0)
    y = jnp.dot(h, w_ref[...], preferred_element_type=jnp.float32) * dinv
    y_ref[0] = y[:, :128]
    y_ref[1] = y[:, 128:]


def _tc_post_body(s_ref, deg_ref, g3_ref, beta3_ref, wo1_ref, g4_ref,
                  beta4_ref, wo2_ref, bo2_ref, out_ref):
    dinv = _dinv_of(deg_ref)
    s_cat = jnp.concatenate([s_ref[0], s_ref[1]], axis=1)
    h = jnp.maximum(s_cat * dinv * g3_ref[...] + beta3_ref[...], 0.0)
    h4 = jnp.dot(h, wo1_ref[...], preferred_element_type=jnp.float32)
    h4 = jnp.maximum(h4 * g4_ref[...] + beta4_ref[...], 0.0)
    lg = jnp.dot(h4, wo2_ref[...], preferred_element_type=jnp.float32)
    lg = lg + bo2_ref[...]
    m = jnp.max(lg, axis=1, keepdims=True)
    lse = m + jnp.log(jnp.sum(jnp.exp(lg - m), axis=1, keepdims=True))
    out_ref[...] = lg - lse


def _row(r, h):
    return pl.BlockSpec((r, h), lambda i: (i, 0))


def _fixed(*shape):
    nd = len(shape)
    return pl.BlockSpec(shape, lambda i, _n=nd: (0,) * _n)


def _half(r):
    return pl.BlockSpec((NC, r, 128), lambda i: (0, i, 0))


def _degspec(r):
    return pl.BlockSpec((NC, r, 16), lambda i: (0, i, 0))


def kernel(x, edge_index, Wp, bp, W1, b1, W2, b2, W3, b3, Wo1, bo1, Wo2, bo2,
           gn1, bnb1, gn2, bnb2, gn3, bnb3, gn4, bnb4):
    n, f_in = x.shape
    h_dim = W1.shape[0]
    e = edge_index.shape[1]
    c_dim = Wo2.shape[1]
    assert n % 8 == 0 and h_dim == 256

    e2 = -(-e // (NS * UNIT)) * (NS * UNIT)
    pad = e2 - e
    src_p = jnp.concatenate(
        [edge_index[0], jnp.zeros((pad,), edge_index.dtype)])
    dst_p = jnp.concatenate(
        [edge_index[1], n + (jnp.arange(pad, dtype=edge_index.dtype) % 8)])
    dst3 = dst_p.reshape(e2 // UNIT, 8, GRP)
    u_total = e2 // UNIT
    nu = e2 // (NS * UNIT)
    src5 = src_p.reshape(NS, nu, 8, GRP)
    dst5 = dst_p.reshape(NS, nu, 8, GRP)

    sc = 1.0 / np.sqrt(1.0 + 1e-5)
    g1 = (gn1 * sc).reshape(1, h_dim)
    g2 = (gn2 * sc).reshape(1, h_dim)
    g3 = (gn3 * sc).reshape(1, h_dim)
    g4 = (gn4 * sc).reshape(1, h_dim)
    beta1 = (b1 * g1[0] + bnb1).reshape(1, h_dim)
    beta2 = (b2 * g2[0] + bnb2).reshape(1, h_dim)
    beta3 = (b3 * g3[0] + bnb3).reshape(1, h_dim)
    beta4 = (bo1 * g4[0] + bnb4).reshape(1, h_dim)

    deg = _deg_call(n, u_total)(dst3)

    r = 2000
    grid = (n // r,)

    y1 = pl.pallas_call(
        _tc_pre_body,
        grid=grid,
        in_specs=[_row(r, f_in), _fixed(f_in, h_dim), _fixed(1, h_dim),
                  _fixed(h_dim, h_dim), _degspec(r)],
        out_specs=_half(r),
        out_shape=jax.ShapeDtypeStruct((NC, n, 128), jnp.float32),
    )(x, Wp, bp.reshape(1, h_dim), W1, deg)

    prop = _prop_call(n, nu)
    s1 = prop(y1, src5, dst5)

    def mid(s_in, g, beta, w):
        return pl.pallas_call(
            _tc_mid_body,
            grid=grid,
            in_specs=[_half(r), _degspec(r), _fixed(1, h_dim),
                      _fixed(1, h_dim), _fixed(h_dim, h_dim)],
            out_specs=_half(r),
            out_shape=jax.ShapeDtypeStruct((NC, n, 128), jnp.float32),
        )(s_in, deg, g, beta, w)

    y2 = mid(s1, g1, beta1, W2)
    s2 = prop(y2, src5, dst5)
    y3 = mid(s2, g2, beta2, W3)
    s3 = prop(y3, src5, dst5)

    out = pl.pallas_call(
        _tc_post_body,
        grid=grid,
        in_specs=[_half(r), _degspec(r), _fixed(1, h_dim), _fixed(1, h_dim),
                  _fixed(h_dim, h_dim), _fixed(1, h_dim), _fixed(1, h_dim),
                  _fixed(h_dim, c_dim), _fixed(1, c_dim)],
        out_specs=_row(r, c_dim),
        out_shape=jax.ShapeDtypeStruct((n, c_dim), jnp.float32),
    )(s3, deg, g3, beta3, Wo1, g4, beta4, Wo2, bo2.reshape(1, c_dim))
    return out


# speedup vs baseline: 1.5018x; 1.2501x over previous
import functools

import jax
import jax.numpy as jnp
import numpy as np
from jax import lax
from jax.experimental import pallas as pl
from jax.experimental.pallas import tpu as pltpu
from jax.experimental.pallas import tpu_sc as plsc

NC = 2
NS = 16
GRP = 32
UNIT = 8 * GRP


def _mesh():
    return plsc.VectorSubcoreMesh(core_axis_name="c", subcore_axis_name="s")


def _node_split(n, w):
    blocks = n // 8
    q, rr = blocks // NS, blocks % NS
    start = 8 * (w * q + jnp.minimum(w, rr))
    return start, 8 * q, w < rr


def _deg_call(n, u_total):
    u_per = u_total // (NC * NS)

    @functools.partial(
        pl.kernel,
        out_type=jax.ShapeDtypeStruct((NC, n, 16), jnp.float32),
        mesh=_mesh(),
        scratch_types=[
            pltpu.VMEM_SHARED((n + 8, 16), jnp.float32),
            pltpu.VMEM((8 * (n // 8 // NS + 1), 16), jnp.float32),
            pltpu.VMEM((GRP, 16), jnp.float32),
            pltpu.VMEM((8, GRP), jnp.int32),
        ],
    )
    def deg_kernel(dst_hbm, out_hbm, acc, zbuf, ones, idx):
        c = lax.axis_index("c")
        s = lax.axis_index("s")
        w = c * NS + s
        zrows = zbuf.shape[0]

        def fill(i, _):
            zbuf[i] = jnp.zeros((16,), jnp.float32)
            return 0

        lax.fori_loop(0, zrows, fill, 0)

        def fill1(i, _):
            ones[i] = jnp.ones((16,), jnp.float32)
            return 0

        lax.fori_loop(0, GRP, fill1, 0)

        start, main, extra = _node_split(n, s)
        pltpu.sync_copy(zbuf.at[pl.ds(0, main)], acc.at[pl.ds(start, main)])

        @pl.when(extra)
        def _():
            pltpu.sync_copy(zbuf.at[pl.ds(0, 8)],
                            acc.at[pl.ds(start + main, 8)])

        @pl.when(s == 0)
        def _():
            pltpu.sync_copy(zbuf.at[pl.ds(0, 8)], acc.at[pl.ds(n, 8)])

        plsc.subcore_barrier()

        def body(i, _):
            pltpu.sync_copy(dst_hbm.at[w * u_per + i], idx)
            for j in range(8):
                pltpu.sync_copy(ones, acc.at[idx.at[j]], add=True)
            return 0

        lax.fori_loop(0, u_per, body, 0)
        plsc.subcore_barrier()
        pltpu.sync_copy(acc.at[pl.ds(start, main)],
                        out_hbm.at[c, pl.ds(start, main)])

        @pl.when(extra)
        def _():
            pltpu.sync_copy(acc.at[pl.ds(start + main, 8)],
                            out_hbm.at[c, pl.ds(start + main, 8)])

    return deg_kernel


NSLOT = 8
AHEAD = 6


def _prop_call(n, nu):
    @functools.partial(
        pl.kernel,
        out_type=jax.ShapeDtypeStruct((NC, n, 128), jnp.float32),
        mesh=_mesh(),
        scratch_types=[
            pltpu.VMEM_SHARED((n + 8, 128), jnp.float32),
            pltpu.VMEM((2, 8, GRP), jnp.int32),
            pltpu.VMEM((2, 8, GRP), jnp.int32),
            [pltpu.VMEM((GRP, 128), jnp.float32)] * NSLOT,
            [pltpu.SemaphoreType.DMA] * NSLOT,
            [pltpu.SemaphoreType.DMA] * NSLOT,
            pltpu.SemaphoreType.DMA,
            pltpu.SemaphoreType.DMA,
        ],
    )
    def prop_kernel(y_hbm, src_hbm, dst_hbm, out_hbm,
                    acc, idx_s, idx_d, rows, sg, ss, sis, sid):
        c = lax.axis_index("c")
        s = lax.axis_index("s")
        g_total = 8 * nu
        start, main, extra = _node_split(n, s)
        pltpu.sync_copy(y_hbm.at[c, pl.ds(start, main)],
                        acc.at[pl.ds(start, main)])

        @pl.when(extra)
        def _():
            pltpu.sync_copy(y_hbm.at[c, pl.ds(start + main, 8)],
                            acc.at[pl.ds(start + main, 8)])

        pltpu.sync_copy(src_hbm.at[s, 0], idx_s.at[0])
        pltpu.sync_copy(dst_hbm.at[s, 0], idx_d.at[0])
        plsc.subcore_barrier()

        def gather(p, j, slot):
            pltpu.async_copy(y_hbm.at[c].at[idx_s.at[p, j]], rows[slot],
                             sg[slot])

        def wait_gather(p, j, slot):
            pltpu.make_async_copy(y_hbm.at[c].at[idx_s.at[p, j]], rows[slot],
                                  sg[slot]).wait()

        def wait_scatter(p, j, slot):
            pltpu.make_async_copy(rows[slot], acc.at[idx_d.at[p, j]],
                                  ss[slot]).wait()

        for g in range(AHEAD):
            gather(0, g, g)

        def body(u, _):
            p = u % 2

            @pl.when(u + 1 < nu)
            def _():
                pltpu.async_copy(src_hbm.at[s, u + 1], idx_s.at[1 - p], sis)
                pltpu.async_copy(dst_hbm.at[s, u + 1], idx_d.at[1 - p], sid)

            for j in range(8):
                g = 8 * u + j
                k = j % NSLOT
                k2 = (j + AHEAD) % NSLOT
                wait_gather(p, j, k)
                pltpu.async_copy(rows[k], acc.at[idx_d.at[p, j]], ss[k],
                                 add=True)

                @pl.when(g >= NSLOT - AHEAD)
                def _():
                    wait_scatter(p, j, k2)

                ja = j + AHEAD
                if ja >= 8:
                    @pl.when(u + 1 < nu)
                    def _():
                        if ja == 8:
                            pltpu.make_async_copy(src_hbm.at[s, u],
                                                  idx_s.at[p], sis).wait()
                            pltpu.make_async_copy(dst_hbm.at[s, u],
                                                  idx_d.at[p], sid).wait()
                        gather(1 - p, ja - 8, k2)
                else:
                    @pl.when(g + AHEAD < g_total)
                    def _():
                        gather(p, ja, k2)
            return 0

        lax.fori_loop(0, nu, body, 0)
        for i in range(NSLOT - AHEAD):
            slot = (g_total - (NSLOT - AHEAD) + i) % NSLOT
            wait_scatter((nu - 1) % 2, 0, slot)
        plsc.subcore_barrier()
        pltpu.sync_copy(acc.at[pl.ds(start, main)],
                        out_hbm.at[c, pl.ds(start, main)])

        @pl.when(extra)
        def _():
            pltpu.sync_copy(acc.at[pl.ds(start + main, 8)],
                            out_hbm.at[c, pl.ds(start + main, 8)])

    return prop_kernel


def _dinv_of(deg_ref):
    d = deg_ref[0, :, 0:1] + deg_ref[1, :, 0:1] + 1.0
    return lax.rsqrt(d)


def _tc_pre_body(x_ref, wp_ref, bp_ref, w1_ref, deg_ref, y_ref):
    h0 = jnp.dot(x_ref[...], wp_ref[...], preferred_element_type=jnp.float32)
    h0 = jnp.maximum(h0 + bp_ref[...], 0.0)
    dinv = _dinv_of(deg_ref)
    y = jnp.dot(h0, w1_ref[...], preferred_element_type=jnp.float32) * dinv
    y_ref[0] = y[:, :128]
    y_ref[1] = y[:, 128:]


def _tc_mid_body(s_ref, deg_ref, g_ref, beta_ref, w_ref, y_ref):
    dinv = _dinv_of(deg_ref)
    s_cat = jnp.concatenate([s_ref[0], s_ref[1]], axis=1)
    h = jnp.maximum(s_cat * dinv * g_ref[...] + beta_ref[...], 0.0)
    y = jnp.dot(h, w_ref[...], preferred_element_type=jnp.float32) * dinv
    y_ref[0] = y[:, :128]
    y_ref[1] = y[:, 128:]


def _tc_post_body(s_ref, deg_ref, g3_ref, beta3_ref, wo1_ref, g4_ref,
                  beta4_ref, wo2_ref, bo2_ref, out_ref):
    dinv = _dinv_of(deg_ref)
    s_cat = jnp.concatenate([s_ref[0], s_ref[1]], axis=1)
    h = jnp.maximum(s_cat * dinv * g3_ref[...] + beta3_ref[...], 0.0)
    h4 = jnp.dot(h, wo1_ref[...], preferred_element_type=jnp.float32)
    h4 = jnp.maximum(h4 * g4_ref[...] + beta4_ref[...], 0.0)
    lg = jnp.dot(h4, wo2_ref[...], preferred_element_type=jnp.float32)
    lg = lg + bo2_ref[...]
    m = jnp.max(lg, axis=1, keepdims=True)
    lse = m + jnp.log(jnp.sum(jnp.exp(lg - m), axis=1, keepdims=True))
    out_ref[...] = lg - lse


def _row(r, h):
    return pl.BlockSpec((r, h), lambda i: (i, 0))


def _fixed(*shape):
    nd = len(shape)
    return pl.BlockSpec(shape, lambda i, _n=nd: (0,) * _n)


def _half(r):
    return pl.BlockSpec((NC, r, 128), lambda i: (0, i, 0))


def _degspec(r):
    return pl.BlockSpec((NC, r, 16), lambda i: (0, i, 0))


def kernel(x, edge_index, Wp, bp, W1, b1, W2, b2, W3, b3, Wo1, bo1, Wo2, bo2,
           gn1, bnb1, gn2, bnb2, gn3, bnb3, gn4, bnb4):
    n, f_in = x.shape
    h_dim = W1.shape[0]
    e = edge_index.shape[1]
    c_dim = Wo2.shape[1]
    assert n % 8 == 0 and h_dim == 256

    e2 = -(-e // (NS * UNIT)) * (NS * UNIT)
    pad = e2 - e
    src_p = jnp.concatenate(
        [edge_index[0], jnp.zeros((pad,), edge_index.dtype)])
    dst_p = jnp.concatenate(
        [edge_index[1], n + (jnp.arange(pad, dtype=edge_index.dtype) % 8)])
    dst3 = dst_p.reshape(e2 // UNIT, 8, GRP)
    u_total = e2 // UNIT
    nu = e2 // (NS * UNIT)
    src5 = src_p.reshape(NS, nu, 8, GRP)
    dst5 = dst_p.reshape(NS, nu, 8, GRP)

    sc = 1.0 / np.sqrt(1.0 + 1e-5)
    g1 = (gn1 * sc).reshape(1, h_dim)
    g2 = (gn2 * sc).reshape(1, h_dim)
    g3 = (gn3 * sc).reshape(1, h_dim)
    g4 = (gn4 * sc).reshape(1, h_dim)
    beta1 = (b1 * g1[0] + bnb1).reshape(1, h_dim)
    beta2 = (b2 * g2[0] + bnb2).reshape(1, h_dim)
    beta3 = (b3 * g3[0] + bnb3).reshape(1, h_dim)
    beta4 = (bo1 * g4[0] + bnb4).reshape(1, h_dim)

    deg = _deg_call(n, u_total)(dst3)

    r = 2000
    grid = (n // r,)

    y1 = pl.pallas_call(
        _tc_pre_body,
        grid=grid,
        in_specs=[_row(r, f_in), _fixed(f_in, h_dim), _fixed(1, h_dim),
                  _fixed(h_dim, h_dim), _degspec(r)],
        out_specs=_half(r),
        out_shape=jax.ShapeDtypeStruct((NC, n, 128), jnp.float32),
    )(x, Wp, bp.reshape(1, h_dim), W1, deg)

    prop = _prop_call(n, nu)
    s1 = prop(y1, src5, dst5)

    def mid(s_in, g, beta, w):
        return pl.pallas_call(
            _tc_mid_body,
            grid=grid,
            in_specs=[_half(r), _degspec(r), _fixed(1, h_dim),
                      _fixed(1, h_dim), _fixed(h_dim, h_dim)],
            out_specs=_half(r),
            out_shape=jax.ShapeDtypeStruct((NC, n, 128), jnp.float32),
        )(s_in, deg, g, beta, w)

    y2 = mid(s1, g1, beta1, W2)
    s2 = prop(y2, src5, dst5)
    y3 = mid(s2, g2, beta2, W3)
    s3 = prop(y3, src5, dst5)

    out = pl.pallas_call(
        _tc_post_body,
        grid=grid,
        in_specs=[_half(r), _degspec(r), _fixed(1, h_dim), _fixed(1, h_dim),
                  _fixed(h_dim, h_dim), _fixed(1, h_dim), _fixed(1, h_dim),
                  _fixed(h_dim, c_dim), _fixed(1, c_dim)],
        out_specs=_row(r, c_dim),
        out_shape=jax.ShapeDtypeStruct((n, c_dim), jnp.float32),
    )(s3, deg, g3, beta3, Wo1, g4, beta4, Wo2, bo2.reshape(1, c_dim))
    return out
